# baseline (speedup 1.0000x reference)
import functools

import jax
import jax.numpy as jnp
from jax.experimental import pallas as pl
from jax.experimental.pallas import tpu as pltpu

NUM_LABELED = 15080
OUT_CHANNELS = 2048
TEMP = 0.05
BATCH = 64

NSTREAM = 5
TILE = 232
NTILES = NUM_LABELED // (NSTREAM * TILE)


def _ce_body(feat_ref, tgt_ref, *refs):
    mem_refs = refs[:NSTREAM]
    out_ref = refs[NSTREAM]
    m_ref, s_ref, p_ref = refs[NSTREAM + 1:]
    t = pl.program_id(0)

    @pl.when(t == 0)
    def _init():
        m_ref[...] = jnp.full((BATCH, 1), -jnp.inf, jnp.float32)
        s_ref[...] = jnp.zeros((BATCH, 1), jnp.float32)
        p_ref[...] = jnp.zeros((BATCH, 1), jnp.float32)

    feat = feat_ref[...]
    logits = jnp.concatenate(
        [jax.lax.dot_general(feat, mr[...], (((1,), (1,)), ((), ())),
                             preferred_element_type=jnp.float32)
         for mr in mem_refs], axis=1)

    base = jax.lax.broadcasted_iota(jnp.int32, (BATCH, NSTREAM * TILE), 1)
    col = (NSTREAM * t + base // TILE) * TILE + base % TILE

    m_old = m_ref[...]
    m_new = jnp.maximum(m_old, jnp.max(logits, axis=1, keepdims=True))
    e = jnp.exp(logits - m_new)
    s_ref[...] = s_ref[...] * jnp.exp(m_old - m_new) + jnp.sum(
        e, axis=1, keepdims=True)
    m_ref[...] = m_new

    hit = col == tgt_ref[...]
    p_ref[...] += jnp.sum(jnp.where(hit, logits, 0.0), axis=1, keepdims=True)

    @pl.when(t == NTILES - 1)
    def _fini():
        lse = m_ref[...] + jnp.log(s_ref[...])
        out_ref[0, 0] = jnp.mean(lse - p_ref[...])


@functools.partial(jax.jit, static_argnames=("interpret",))
def _ce_loss(feat, targets, memory_features, interpret=False):
    def mk_map(i):
        return lambda t: (NSTREAM * t + i, 0)

    out = pl.pallas_call(
        _ce_body,
        grid=(NTILES,),
        in_specs=[
            pl.BlockSpec((BATCH, OUT_CHANNELS), lambda t: (0, 0)),
            pl.BlockSpec((BATCH, 1), lambda t: (0, 0)),
        ] + [pl.BlockSpec((TILE, OUT_CHANNELS), mk_map(i))
             for i in range(NSTREAM)],
        out_specs=pl.BlockSpec(memory_space=pltpu.SMEM),
        out_shape=jax.ShapeDtypeStruct((1, 1), jnp.float32),
        scratch_shapes=[
            pltpu.VMEM((BATCH, 1), jnp.float32),
            pltpu.VMEM((BATCH, 1), jnp.float32),
            pltpu.VMEM((BATCH, 1), jnp.float32),
        ],
        interpret=interpret,
    )(feat, targets, *([memory_features] * NSTREAM))
    return out[0, 0]


def kernel(features, features_k, gt_labels, gt_labels_k, memory_features):
    pids = gt_labels[0, :, -1]
    mask = pids > -1
    feat = jnp.where(mask[:, None], features / TEMP, 0.0)
    targets = jnp.where(mask, pids, 0).astype(jnp.int32)[:, None]
    return _ce_loss(feat, targets, memory_features)
